# Initial kernel scaffold; baseline (speedup 1.0000x reference)
import functools

import jax
import jax.numpy as jnp
from jax import lax
from jax.experimental import pallas as pl
from jax.experimental.pallas import tpu as pltpu
from jax.experimental.pallas import tpu_sc as plsc

N_NODES = 10000
N_EDGES = 160000
IN_DIM = 256
OUT_DIM = 256

NC = 2
NS = 16
HALF = IN_DIM // 2
CHUNK = 128
CHUNKS_PER_TILE = 79
E_PAD = NS * CHUNKS_PER_TILE * CHUNK
AGG_ROWS = 10240
ROWS_PER_TILE_ZERO = AGG_ROWS // NS
ROWS_PER_TILE_OUT = N_NODES // NS


def _sc_agg_body(f0, f1, srcs, dsts, zrows, out, src_v, dst_v, rows_v,
                 agg_sh, sem):
    cid = lax.axis_index("c")
    sid = lax.axis_index("s")

    pltpu.sync_copy(zrows, agg_sh.at[pl.ds(sid * ROWS_PER_TILE_ZERO,
                                           ROWS_PER_TILE_ZERO)])
    plsc.subcore_barrier()

    pltpu.sync_copy(srcs.at[pl.ds(sid * CHUNKS_PER_TILE, CHUNKS_PER_TILE)],
                    src_v)
    pltpu.sync_copy(dsts.at[pl.ds(sid * CHUNKS_PER_TILE, CHUNKS_PER_TILE)],
                    dst_v)

    def edge_pass(feat):
        def chunk(j, carry):
            pltpu.async_copy(feat.at[src_v.at[j]], rows_v, sem).wait()
            pltpu.sync_copy(rows_v, agg_sh.at[dst_v.at[j]], add=True)
            return carry
        lax.fori_loop(0, CHUNKS_PER_TILE, chunk, 0)

    pl.when(cid == 0)(lambda: edge_pass(f0))
    pl.when(cid == 1)(lambda: edge_pass(f1))

    plsc.subcore_barrier()

    pltpu.sync_copy(
        agg_sh.at[pl.ds(sid * ROWS_PER_TILE_OUT, ROWS_PER_TILE_OUT)],
        out.at[cid, pl.ds(sid * ROWS_PER_TILE_OUT, ROWS_PER_TILE_OUT)])


_sc_agg = pl.kernel(
    _sc_agg_body,
    out_type=jax.ShapeDtypeStruct((NC, N_NODES, HALF), jnp.float32),
    mesh=plsc.VectorSubcoreMesh(core_axis_name="c", subcore_axis_name="s",
                                num_cores=NC, num_subcores=NS),
    scratch_types=[
        pltpu.VMEM((CHUNKS_PER_TILE, CHUNK), jnp.int32),
        pltpu.VMEM((CHUNKS_PER_TILE, CHUNK), jnp.int32),
        pltpu.VMEM((CHUNK, HALF), jnp.float32),
        pltpu.VMEM_SHARED((AGG_ROWS, HALF), jnp.float32),
        pltpu.SemaphoreType.DMA,
    ],
)


ROW_BLK = 2000
GRID = N_NODES // ROW_BLK


def _tc_body(f_ref, a0_ref, a1_ref, w1_ref, w2a_ref, w2b_ref, w3_ref,
             out_ref, acc_ref):
    i = pl.program_id(0)
    z = jnp.tanh(
        jnp.dot(f_ref[...], w1_ref[...], preferred_element_type=jnp.float32)
        + jnp.dot(a0_ref[...], w2a_ref[...],
                  preferred_element_type=jnp.float32)
        + jnp.dot(a1_ref[...], w2b_ref[...],
                  preferred_element_type=jnp.float32))
    p = jnp.sum(z, axis=0, keepdims=True)

    @pl.when(i == 0)
    def _():
        acc_ref[...] = p

    @pl.when(i != 0)
    def _():
        acc_ref[...] = acc_ref[...] + p

    @pl.when(i == GRID - 1)
    def _():
        out_ref[...] = jnp.dot(acc_ref[...], w3_ref[...],
                               preferred_element_type=jnp.float32)


_tc_pool = pl.pallas_call(
    _tc_body,
    grid=(GRID,),
    in_specs=[
        pl.BlockSpec((ROW_BLK, IN_DIM), lambda i: (i, 0)),
        pl.BlockSpec((ROW_BLK, HALF), lambda i: (i, 0)),
        pl.BlockSpec((ROW_BLK, HALF), lambda i: (i, 0)),
        pl.BlockSpec((IN_DIM, OUT_DIM), lambda i: (0, 0)),
        pl.BlockSpec((HALF, OUT_DIM), lambda i: (0, 0)),
        pl.BlockSpec((HALF, OUT_DIM), lambda i: (0, 0)),
        pl.BlockSpec((OUT_DIM, OUT_DIM), lambda i: (0, 0)),
    ],
    out_specs=pl.BlockSpec((1, OUT_DIM), lambda i: (0, 0)),
    out_shape=jax.ShapeDtypeStruct((1, OUT_DIM), jnp.float32),
    scratch_shapes=[pltpu.VMEM((1, OUT_DIM), jnp.float32)],
)


@jax.jit
def kernel(features, edge_index, W1, W2, W3):
    f0 = features[:, :HALF]
    f1 = features[:, HALF:]

    src = edge_index[0]
    dst = edge_index[1]
    pad = E_PAD - N_EDGES
    src_p = jnp.concatenate(
        [src, jnp.zeros((pad,), jnp.int32)]).reshape(-1, CHUNK)
    dst_p = jnp.concatenate(
        [dst, jnp.full((pad,), N_NODES, jnp.int32)]).reshape(-1, CHUNK)

    zrows = jnp.zeros((ROWS_PER_TILE_ZERO, HALF), jnp.float32)

    agg = _sc_agg(f0, f1, src_p, dst_p, zrows)

    return _tc_pool(features, agg[0], agg[1], W1, W2[:HALF], W2[HALF:], W3)


# speedup vs baseline: 3.0907x; 3.0907x over previous
import functools

import jax
import jax.numpy as jnp
from jax import lax
from jax.experimental import pallas as pl
from jax.experimental.pallas import tpu as pltpu
from jax.experimental.pallas import tpu_sc as plsc

N_NODES = 10000
N_EDGES = 160000
IN_DIM = 256
OUT_DIM = 256

NC = 2
NS = 16
HALF = IN_DIM // 2
CHUNK = 128
CHUNKS_PER_TILE = 80
E_PAD = NS * CHUNKS_PER_TILE * CHUNK
AGG_ROWS = 10240
ROWS_PER_TILE = AGG_ROWS // NS


def _sc_agg_body(f0, f1, srcs, dsts, zrows, out, src_v, dst_v, rows_v,
                 agg_sh, sem):
    cid = lax.axis_index("c")
    sid = lax.axis_index("s")

    pltpu.sync_copy(zrows, agg_sh.at[pl.ds(sid * ROWS_PER_TILE,
                                           ROWS_PER_TILE)])
    plsc.subcore_barrier()

    pltpu.sync_copy(srcs.at[pl.ds(sid * CHUNKS_PER_TILE, CHUNKS_PER_TILE)],
                    src_v)
    pltpu.sync_copy(dsts.at[pl.ds(sid * CHUNKS_PER_TILE, CHUNKS_PER_TILE)],
                    dst_v)

    def edge_pass(feat):
        def chunk(j, carry):
            pltpu.async_copy(feat.at[src_v.at[j]], rows_v, sem).wait()
            pltpu.sync_copy(rows_v, agg_sh.at[dst_v.at[j]], add=True)
            return carry
        lax.fori_loop(0, CHUNKS_PER_TILE, chunk, 0)

    pl.when(cid == 0)(lambda: edge_pass(f0))
    pl.when(cid == 1)(lambda: edge_pass(f1))

    plsc.subcore_barrier()

    pltpu.sync_copy(
        agg_sh.at[pl.ds(sid * ROWS_PER_TILE, ROWS_PER_TILE)],
        out.at[cid, pl.ds(sid * ROWS_PER_TILE, ROWS_PER_TILE)])


_sc_agg = pl.kernel(
    _sc_agg_body,
    out_type=jax.ShapeDtypeStruct((NC, AGG_ROWS, HALF), jnp.float32),
    mesh=plsc.VectorSubcoreMesh(core_axis_name="c", subcore_axis_name="s",
                                num_cores=NC, num_subcores=NS),
    scratch_types=[
        pltpu.VMEM((CHUNKS_PER_TILE, CHUNK), jnp.int32),
        pltpu.VMEM((CHUNKS_PER_TILE, CHUNK), jnp.int32),
        pltpu.VMEM((CHUNK, HALF), jnp.float32),
        pltpu.VMEM_SHARED((AGG_ROWS, HALF), jnp.float32),
        pltpu.SemaphoreType.DMA,
    ],
)


ROW_BLK = 2000
GRID = N_NODES // ROW_BLK


def _tc_body(f_ref, a0_ref, a1_ref, w1_ref, w2a_ref, w2b_ref, w3_ref,
             out_ref, acc_ref):
    i = pl.program_id(0)
    z = jnp.tanh(
        jnp.dot(f_ref[...], w1_ref[...], preferred_element_type=jnp.float32)
        + jnp.dot(a0_ref[...], w2a_ref[...],
                  preferred_element_type=jnp.float32)
        + jnp.dot(a1_ref[...], w2b_ref[...],
                  preferred_element_type=jnp.float32))
    p = jnp.sum(z, axis=0, keepdims=True)

    @pl.when(i == 0)
    def _():
        acc_ref[...] = p

    @pl.when(i != 0)
    def _():
        acc_ref[...] = acc_ref[...] + p

    @pl.when(i == GRID - 1)
    def _():
        out_ref[...] = jnp.dot(acc_ref[...], w3_ref[...],
                               preferred_element_type=jnp.float32)


_tc_pool = pl.pallas_call(
    _tc_body,
    grid=(GRID,),
    in_specs=[
        pl.BlockSpec((ROW_BLK, IN_DIM), lambda i: (i, 0)),
        pl.BlockSpec((ROW_BLK, HALF), lambda i: (i, 0)),
        pl.BlockSpec((ROW_BLK, HALF), lambda i: (i, 0)),
        pl.BlockSpec((IN_DIM, OUT_DIM), lambda i: (0, 0)),
        pl.BlockSpec((HALF, OUT_DIM), lambda i: (0, 0)),
        pl.BlockSpec((HALF, OUT_DIM), lambda i: (0, 0)),
        pl.BlockSpec((OUT_DIM, OUT_DIM), lambda i: (0, 0)),
    ],
    out_specs=pl.BlockSpec((1, OUT_DIM), lambda i: (0, 0)),
    out_shape=jax.ShapeDtypeStruct((1, OUT_DIM), jnp.float32),
    scratch_shapes=[pltpu.VMEM((1, OUT_DIM), jnp.float32)],
)


@jax.jit
def kernel(features, edge_index, W1, W2, W3):
    f0 = features[:, :HALF]
    f1 = features[:, HALF:]

    src = edge_index[0]
    dst = edge_index[1]
    pad = E_PAD - N_EDGES
    src_p = jnp.concatenate(
        [src, jnp.zeros((pad,), jnp.int32)]).reshape(-1, CHUNK)
    dst_p = jnp.concatenate(
        [dst, jnp.full((pad,), N_NODES, jnp.int32)]).reshape(-1, CHUNK)

    zrows = jnp.zeros((ROWS_PER_TILE, HALF), jnp.float32)

    agg = _sc_agg(f0, f1, src_p, dst_p, zrows)

    return _tc_pool(features, agg[0, :N_NODES], agg[1, :N_NODES],
                    W1, W2[:HALF], W2[HALF:], W3)


# speedup vs baseline: 3.6662x; 1.1862x over previous
import functools

import jax
import jax.numpy as jnp
from jax import lax
from jax.experimental import pallas as pl
from jax.experimental.pallas import tpu as pltpu
from jax.experimental.pallas import tpu_sc as plsc

N_NODES = 10000
N_EDGES = 160000
IN_DIM = 256
OUT_DIM = 256

NC = 2
NS = 16
HALF = IN_DIM // 2
CHUNK = 128
CHUNKS_PER_TILE = 80
E_PAD = NS * CHUNKS_PER_TILE * CHUNK
AGG_ROWS = 10008
TRASH_ROW = N_NODES

IDX_GROUP = 32
GROUPS = [(0, 32), (32, 32), (64, 16)]

SLAB = 632
LAST_SLAB = N_NODES - 15 * SLAB
LAST_BASE = 15 * SLAB


def _sc_agg_body(f0, f1, srcs, dsts, zrows, out,
                 srcA, srcB, dstA, dstB, rA, rB, agg_sh,
                 gA, gB, sA, sB, iS):
    rows_b = [rA, rB]
    gsems = [gA, gB]
    ssems = [sA, sB]
    src_pair = [srcA, srcB]
    dst_pair = [dstA, dstB]
    cid = lax.axis_index("c")
    sid = lax.axis_index("s")
    tbase = sid * CHUNKS_PER_TILE

    def load_idx(gi):
        gbase, gsize = GROUPS[gi]
        p = gi % 2
        pltpu.async_copy(srcs.at[pl.ds(tbase + gbase, gsize)],
                         src_pair[p].at[pl.ds(0, gsize)], iS)
        pltpu.async_copy(dsts.at[pl.ds(tbase + gbase, gsize)],
                         dst_pair[p].at[pl.ds(0, gsize)], iS)

    def wait_idx(gi):
        gbase, gsize = GROUPS[gi]
        p = gi % 2
        pltpu.make_async_copy(srcs.at[pl.ds(tbase + gbase, gsize)],
                              src_pair[p].at[pl.ds(0, gsize)], iS).wait()
        pltpu.make_async_copy(dsts.at[pl.ds(tbase + gbase, gsize)],
                              dst_pair[p].at[pl.ds(0, gsize)], iS).wait()

    load_idx(0)

    @pl.when(sid < 15)
    def _():
        pltpu.sync_copy(zrows, agg_sh.at[pl.ds(sid * SLAB, SLAB)])

    @pl.when(sid == 15)
    def _():
        pltpu.sync_copy(zrows.at[pl.ds(0, LAST_SLAB)],
                        agg_sh.at[pl.ds(LAST_BASE, LAST_SLAB)])

    wait_idx(0)
    plsc.subcore_barrier()

    def edge_pass(feat):
        def start_gather(sv, c, b):
            pltpu.async_copy(feat.at[sv.at[c]], rows_b[b], gsems[b])

        def wait_gather(sv, b):
            pltpu.make_async_copy(feat.at[sv.at[0]], rows_b[b],
                                  gsems[b]).wait()

        def start_scatter(dv, c, b):
            pltpu.async_copy(rows_b[b], agg_sh.at[dv.at[c]], ssems[b],
                             add=True)

        def wait_scatter(dv, b):
            pltpu.make_async_copy(rows_b[b], agg_sh.at[dv.at[0]],
                                  ssems[b]).wait()

        for gi, (gbase, gsize) in enumerate(GROUPS):
            sv = src_pair[gi % 2]
            dv = dst_pair[gi % 2]
            if gi + 1 < len(GROUPS):
                load_idx(gi + 1)
            start_gather(sv, 0, 0)
            start_gather(sv, 1, 1)

            def step(jo, carry):
                for b in range(2):
                    c = jo * 2 + b
                    wait_gather(sv, b)
                    start_scatter(dv, c, b)

                    @pl.when(c + 2 < gsize)
                    def _():
                        wait_scatter(dv, b)
                        start_gather(sv, c + 2, b)
                return carry

            lax.fori_loop(0, gsize // 2, step, 0)
            for b in range(2):
                wait_scatter(dv, b)
            if gi + 1 < len(GROUPS):
                wait_idx(gi + 1)

    pl.when(cid == 0)(lambda: edge_pass(f0))
    pl.when(cid == 1)(lambda: edge_pass(f1))

    plsc.subcore_barrier()

    @pl.when(sid < 15)
    def _():
        pltpu.sync_copy(agg_sh.at[pl.ds(sid * SLAB, SLAB)],
                        out.at[cid, pl.ds(sid * SLAB, SLAB)])

    @pl.when(sid == 15)
    def _():
        pltpu.sync_copy(agg_sh.at[pl.ds(LAST_BASE, LAST_SLAB)],
                        out.at[cid, pl.ds(LAST_BASE, LAST_SLAB)])


_sc_agg = pl.kernel(
    _sc_agg_body,
    out_type=jax.ShapeDtypeStruct((NC, N_NODES, HALF), jnp.float32),
    mesh=plsc.VectorSubcoreMesh(core_axis_name="c", subcore_axis_name="s",
                                num_cores=NC, num_subcores=NS),
    scratch_types=[
        pltpu.VMEM((IDX_GROUP, CHUNK), jnp.int32),
        pltpu.VMEM((IDX_GROUP, CHUNK), jnp.int32),
        pltpu.VMEM((IDX_GROUP, CHUNK), jnp.int32),
        pltpu.VMEM((IDX_GROUP, CHUNK), jnp.int32),
        pltpu.VMEM((CHUNK, HALF), jnp.float32),
        pltpu.VMEM((CHUNK, HALF), jnp.float32),
        pltpu.VMEM_SHARED((AGG_ROWS, HALF), jnp.float32),
    ] + [pltpu.SemaphoreType.DMA] * 5,
)


ROW_BLK = 2000
GRID = N_NODES // ROW_BLK


def _tc_body(f_ref, a0_ref, a1_ref, w1_ref, w2a_ref, w2b_ref, w3_ref,
             out_ref, acc_ref):
    i = pl.program_id(0)
    z = jnp.tanh(
        jnp.dot(f_ref[...], w1_ref[...], preferred_element_type=jnp.float32)
        + jnp.dot(a0_ref[...], w2a_ref[...],
                  preferred_element_type=jnp.float32)
        + jnp.dot(a1_ref[...], w2b_ref[...],
                  preferred_element_type=jnp.float32))
    p = jnp.sum(z, axis=0, keepdims=True)

    @pl.when(i == 0)
    def _():
        acc_ref[...] = p

    @pl.when(i != 0)
    def _():
        acc_ref[...] = acc_ref[...] + p

    @pl.when(i == GRID - 1)
    def _():
        out_ref[...] = jnp.dot(acc_ref[...], w3_ref[...],
                               preferred_element_type=jnp.float32)


_tc_pool = pl.pallas_call(
    _tc_body,
    grid=(GRID,),
    in_specs=[
        pl.BlockSpec((ROW_BLK, IN_DIM), lambda i: (i, 0)),
        pl.BlockSpec((ROW_BLK, HALF), lambda i: (i, 0)),
        pl.BlockSpec((ROW_BLK, HALF), lambda i: (i, 0)),
        pl.BlockSpec((IN_DIM, OUT_DIM), lambda i: (0, 0)),
        pl.BlockSpec((HALF, OUT_DIM), lambda i: (0, 0)),
        pl.BlockSpec((HALF, OUT_DIM), lambda i: (0, 0)),
        pl.BlockSpec((OUT_DIM, OUT_DIM), lambda i: (0, 0)),
    ],
    out_specs=pl.BlockSpec((1, OUT_DIM), lambda i: (0, 0)),
    out_shape=jax.ShapeDtypeStruct((1, OUT_DIM), jnp.float32),
    scratch_shapes=[pltpu.VMEM((1, OUT_DIM), jnp.float32)],
)


@jax.jit
def kernel(features, edge_index, W1, W2, W3):
    f0 = features[:, :HALF]
    f1 = features[:, HALF:]

    src = edge_index[0]
    dst = edge_index[1]
    pad = E_PAD - N_EDGES
    src_p = jnp.concatenate(
        [src, jnp.zeros((pad,), jnp.int32)]).reshape(-1, CHUNK)
    dst_p = jnp.concatenate(
        [dst, jnp.full((pad,), TRASH_ROW, jnp.int32)]).reshape(-1, CHUNK)

    zrows = jnp.zeros((SLAB, HALF), jnp.float32)

    agg = _sc_agg(f0, f1, src_p, dst_p, zrows)

    return _tc_pool(features, agg[0], agg[1], W1, W2[:HALF], W2[HALF:], W3)
